# baseline (speedup 1.0000x reference)
import jax
import jax.numpy as jnp
from jax import lax
from jax.experimental import pallas as pl
from jax.experimental.pallas import tpu as pltpu
from jax.experimental.pallas import tpu_sc as plsc

_N = 100000
_S = 16
_B = 8
_N2 = _N // 2

_SC_MESH = plsc.VectorSubcoreMesh(core_axis_name="c", subcore_axis_name="s")


def _make_wcat(Wb):
    s, _, co = Wb.shape
    eye = jnp.eye(16, dtype=Wb.dtype)
    k = eye[None, :, None, :, None] * Wb[:, None, :, None, :]
    return k.reshape(s, 256, 16 * co).transpose(1, 0, 2).reshape(256, s * 16 * co)


def _make_wcat_cb(Wb, co_out):
    s, ci, co = Wb.shape
    e8 = jnp.eye(8, dtype=Wb.dtype)
    k = Wb[:, :, None, None, :] * e8[None, None, :, :, None]
    k = k.reshape(s, ci * 8, 8 * co)
    e2 = jnp.eye(2, dtype=Wb.dtype)
    k2 = (e2[None, :, None, :, None] * k[:, None, :, None, :])
    k2 = k2.reshape(s, 2 * ci * 8, 2 * 8 * co)
    assert co == co_out
    return k2.transpose(1, 0, 2).reshape(2 * ci * 8, s * 16 * co)


def _mm_body(h_ref, w_ref, o_ref):
    o_ref[...] = jnp.dot(h_ref[...].astype(w_ref.dtype), w_ref[...],
                         preferred_element_type=jnp.float32).astype(o_ref.dtype)


def _mm2_body(h_ref, g_ref, w_ref, wx_ref, o_ref, ox_ref):
    o_ref[...] = jnp.dot(h_ref[...].astype(w_ref.dtype), w_ref[...],
                         preferred_element_type=jnp.float32)
    ox_ref[...] = jnp.dot(g_ref[...].astype(wx_ref.dtype), wx_ref[...],
                          preferred_element_type=jnp.float32)


_R = 400


def _pair_perm(ko):
    g = jnp.arange(ko) // 32 * 32
    r = jnp.arange(ko) % 32
    return g + jnp.where(r % 2 == 0, r // 2, r // 2 + 16)


def _transform(h2, wc, out_dtype=jnp.float32):
    kk, ko = wc.shape
    return pl.pallas_call(
        _mm_body,
        grid=(_N2 // _R,),
        in_specs=[pl.BlockSpec((_R, kk), lambda i: (i, 0)),
                  pl.BlockSpec((kk, ko), lambda i: (0, 0))],
        out_specs=pl.BlockSpec((_R, ko), lambda i: (i, 0)),
        out_shape=jax.ShapeDtypeStruct((_N2, ko), out_dtype),
    )(h2, wc)


def _transform2(h2, g2, wc, wxc):
    ko, kx = wc.shape[1], wxc.shape[1]
    kg = wxc.shape[0]
    return pl.pallas_call(
        _mm2_body,
        grid=(_N2 // _R,),
        in_specs=[pl.BlockSpec((_R, 256), lambda i: (i, 0)),
                  pl.BlockSpec((_R, kg), lambda i: (i, 0)),
                  pl.BlockSpec((256, ko), lambda i: (0, 0)),
                  pl.BlockSpec((kg, kx), lambda i: (0, 0))],
        out_specs=[pl.BlockSpec((_R, ko), lambda i: (i, 0)),
                   pl.BlockSpec((_R, kx), lambda i: (i, 0))],
        out_shape=[jax.ShapeDtypeStruct((_N2, ko), jnp.float32),
                   jax.ShapeDtypeStruct((_N2, kx), jnp.float32)],
    )(h2, g2, wc, wxc)


_CH = 5
_SUP = 32
_SPN = _CH * _SUP


def _gather_sum(table, sp, bias, xw=None, packed=False):
    rw = table.shape[1]
    ow = 2 * rw if packed else rw
    rc = rw // 16
    gsup_n = _N // _SPN
    nsup = (gsup_n + 31) // 32
    nch = nsup * _SUP
    gl = _CH * 16
    have_xw = xw is not None

    def body(*refs):
        if have_xw:
            (t_h, sp_h, b_h, xw_h, out_h, sp_v, idx_v, g_v, out_v, bias_v,
             xw_v, s_sp0, s_sp1, s_x0, s_x1, s_g0, s_g1, s_o0, s_o1) = refs
        else:
            (t_h, sp_h, b_h, out_h, sp_v, idx_v, g_v, out_v, bias_v,
             s_sp0, s_sp1, s_g0, s_g1, s_o0, s_o1) = refs
            s_x0 = s_x1 = xw_v = None
        wid = lax.axis_index("s") * 2 + lax.axis_index("c")
        pltpu.sync_copy(b_h, bias_v)
        lane2 = lax.iota(jnp.int32, 16) * 2

        def stage_in(si_slot, gsup, sem):
            pltpu.async_copy(sp_h.at[pl.ds(gsup * _SPN, _SPN)],
                             sp_v.at[pl.ds((si_slot % 2) * _SPN, _SPN)], sem)

        def wait_in(sem, dof):
            pltpu.make_async_copy(sp_h.at[pl.ds(0, _SPN)],
                                  sp_v.at[pl.ds(dof, _SPN)], sem).wait()

        def stage_xw(si_slot, gsup, sem):
            pltpu.async_copy(xw_h.at[pl.ds(gsup * _SPN, _SPN)],
                             xw_v.at[pl.ds((si_slot % 2) * _SPN, _SPN)], sem)

        def wait_xw(sem, dof):
            pltpu.make_async_copy(xw_h.at[pl.ds(0, _SPN)],
                                  xw_v.at[pl.ds(dof, _SPN)], sem).wait()

        stage_in(0, wid, s_sp0)
        wait_in(s_sp0, 0)
        stage_in(1, 32 + wid, s_sp1)
        if have_xw:
            stage_xw(0, wid, s_x0)
            wait_xw(s_x0, 0)
            stage_xw(1, 32 + wid, s_x1)

        def step(t, carry):
            si, tin = t // _SUP, t % _SUP
            gb = t % 2
            gsup = si * 32 + wid

            @pl.when((t < nch) & (gsup < gsup_n))
            def _():
                @pl.when((tin == 0) & (si >= 1))
                def _():
                    @pl.when(si % 2 == 0)
                    def _():
                        wait_in(s_sp0, 0)

                    @pl.when(si % 2 == 1)
                    def _():
                        wait_in(s_sp1, _SPN)

                    @pl.when((si + 1 < nsup) & ((si + 1) * 32 + wid < gsup_n))
                    def _():
                        @pl.when((si + 1) % 2 == 0)
                        def _():
                            stage_in(0, (si + 1) * 32 + wid, s_sp0)

                        @pl.when((si + 1) % 2 == 1)
                        def _():
                            stage_in(1, (si + 1) * 32 + wid, s_sp1)

                if have_xw:
                    @pl.when((tin == 1) & (si >= 1) &
                             (si + 1 < nsup) &
                             ((si + 1) * 32 + wid < gsup_n))
                    def _():
                        @pl.when((si + 1) % 2 == 0)
                        def _():
                            stage_xw(0, (si + 1) * 32 + wid, s_x0)

                        @pl.when((si + 1) % 2 == 1)
                        def _():
                            stage_xw(1, (si + 1) * 32 + wid, s_x1)

                srow = (si % 2) * _SPN + tin * _CH
                for j in range(_CH):
                    v = sp_v[srow + j]
                    idx_v[gb, pl.ds(j * 16, 16)] = \
                        ((v >> 1) << 5) + (v & 1) + lane2

                @pl.when(gb == 0)
                def _():
                    pltpu.async_copy(t_h.at[idx_v.at[0]],
                                     g_v.at[pl.ds(0, gl)], s_g0)

                @pl.when(gb == 1)
                def _():
                    pltpu.async_copy(t_h.at[idx_v.at[1]],
                                     g_v.at[pl.ds(gl, gl)], s_g1)

            u = t - 1
            usi, uin = u // _SUP, u % _SUP
            ugsup = usi * 32 + wid

            @pl.when((t >= 1) & (ugsup < gsup_n))
            def _():
                ub = u % 2
                ob = usi % 2

                @pl.when(ub == 0)
                def _():
                    pltpu.make_async_copy(t_h.at[idx_v.at[0]],
                                          g_v.at[pl.ds(0, gl)], s_g0).wait()

                @pl.when(ub == 1)
                def _():
                    pltpu.make_async_copy(t_h.at[idx_v.at[1]],
                                          g_v.at[pl.ds(gl, gl)], s_g1).wait()

                if have_xw:
                    @pl.when((uin == 0) & (usi >= 1))
                    def _():
                        @pl.when(usi % 2 == 0)
                        def _():
                            wait_xw(s_x0, 0)

                        @pl.when(usi % 2 == 1)
                        def _():
                            wait_xw(s_x1, _SPN)

                gof = ub * gl
                orow = ob * _SPN + uin * _CH
                for j in range(_CH):
                    for c in range(rc):
                        sl = pl.ds(c * 16, 16)
                        if packed:
                            lo = bias_v[pl.ds(c * 32, 16)]
                            hi = bias_v[pl.ds(c * 32 + 16, 16)]
                            for k in range(16):
                                w = g_v[gof + j * 16 + k, sl]
                                lo = lo + lax.bitcast_convert_type(
                                    w << 16, jnp.float32)
                                hi = hi + lax.bitcast_convert_type(
                                    w & jnp.int32(-65536), jnp.float32)
                            out_v[orow + j, pl.ds(c * 32, 16)] = lo
                            out_v[orow + j, pl.ds(c * 32 + 16, 16)] = hi
                        else:
                            acc = bias_v[sl]
                            if have_xw:
                                acc = acc + xw_v[orow + j, sl]
                            for k in range(16):
                                acc = acc + g_v[gof + j * 16 + k, sl]
                            out_v[orow + j, sl] = acc

                @pl.when(uin == _SUP - 1)
                def _():
                    dst = ugsup * _SPN

                    @pl.when(ob == 0)
                    def _():
                        pltpu.sync_copy(out_v.at[pl.ds(0, _SPN)],
                                        out_h.at[pl.ds(dst, _SPN)])

                    @pl.when(ob == 1)
                    def _():
                        pltpu.sync_copy(out_v.at[pl.ds(_SPN, _SPN)],
                                        out_h.at[pl.ds(dst, _SPN)])
            return carry

        lax.fori_loop(0, nch + 1, step, 0)

    scratch = [pltpu.VMEM((2 * _SPN, 16), jnp.int32),
               pltpu.VMEM((2, gl), jnp.int32),
               pltpu.VMEM((2 * gl, rw), jnp.int32 if packed else jnp.float32),
               pltpu.VMEM((2 * _SPN, ow), jnp.float32),
               pltpu.VMEM((ow,), jnp.float32)]
    if have_xw:
        scratch.append(pltpu.VMEM((2 * _SPN, rw), jnp.float32))
    nsem = 8 if have_xw else 6
    scratch += [pltpu.SemaphoreType.DMA] * nsem
    args = (table, sp, bias) + ((xw,) if have_xw else ())
    params = (pltpu.CompilerParams(use_tc_tiling_on_sc=False)
              if rw < 128 else None)
    return pl.kernel(body,
                     out_type=jax.ShapeDtypeStruct((_N, ow), jnp.float32),
                     mesh=_SC_MESH,
                     compiler_params=params,
                     scratch_types=scratch)(*args)


def kernel(x, spiral, W0, b0, W1, b1, W2, b2, W_out, b_out):
    sp = spiral.astype(jnp.int32)

    w0b = W0.reshape(_S, 3, 16)
    w1b = W1.reshape(_S, 16, 16)
    wh, wx = W_out[:16, :], W_out[16:, :]
    w2f = (W2 @ wh).reshape(_S, 16, 3)
    w2b = jnp.pad(w2f, ((0, 0), (0, 0), (0, 1)))
    perm = _pair_perm(4096)
    wc0 = _make_wcat_cb(w0b, 16)[:, perm].astype(jnp.bfloat16)
    wc1 = _make_wcat(w1b)[:, perm].astype(jnp.bfloat16)
    wc2 = _make_wcat(w2b).astype(jnp.bfloat16)
    wxp = jnp.pad(wx, ((0, 0), (0, 1)))
    wxc = _make_wcat_cb(wxp[None], 4).astype(jnp.bfloat16)
    bias0 = jnp.tile(b0, (_B,))
    bias1 = jnp.tile(b1, (_B,))
    bias2 = jnp.tile(jnp.pad(b2 @ wh + b_out, (0, 1)), (_B,))

    xt = jnp.transpose(x.reshape(_B, _N * 3)).reshape(_N2, 48)

    def to_i32(t):
        return lax.bitcast_convert_type(
            t.reshape(_N2 * 2048, 2), jnp.int32).reshape(_N2 * 32, 64)

    t0 = to_i32(_transform(xt, wc0, jnp.bfloat16))
    h1 = _gather_sum(t0, sp, bias0, packed=True)
    t1 = to_i32(_transform(h1.reshape(_N2, 256), wc1, jnp.bfloat16))
    h2 = _gather_sum(t1, sp, bias1, packed=True)
    t2, xw = _transform2(h2.reshape(_N2, 256), xt, wc2, wxc)
    outp = _gather_sum(t2.reshape(_N2 * 32, 32), sp, bias2,
                       xw=xw.reshape(_N, 32))

    return jnp.transpose(outp.reshape(_N, _B, 4)[:, :, :3], (1, 0, 2))


# speedup vs baseline: 30.9651x; 30.9651x over previous
import jax
import jax.numpy as jnp
from jax import lax
from jax.experimental import pallas as pl
from jax.experimental.pallas import tpu as pltpu
from jax.experimental.pallas import tpu_sc as plsc

_N = 100000
_S = 16
_B = 8
_N2 = _N // 2

_SC_MESH = plsc.VectorSubcoreMesh(core_axis_name="c", subcore_axis_name="s")


def _make_wcat(Wb):
    s, _, co = Wb.shape
    eye = jnp.eye(16, dtype=Wb.dtype)
    k = eye[None, :, None, :, None] * Wb[:, None, :, None, :]
    return k.reshape(s, 256, 16 * co).transpose(1, 0, 2).reshape(256, s * 16 * co)


def _make_wcat_cb(Wb, co_out):
    s, ci, co = Wb.shape
    e8 = jnp.eye(8, dtype=Wb.dtype)
    k = Wb[:, :, None, None, :] * e8[None, None, :, :, None]
    k = k.reshape(s, ci * 8, 8 * co)
    e2 = jnp.eye(2, dtype=Wb.dtype)
    k2 = (e2[None, :, None, :, None] * k[:, None, :, None, :])
    k2 = k2.reshape(s, 2 * ci * 8, 2 * 8 * co)
    assert co == co_out
    return k2.transpose(1, 0, 2).reshape(2 * ci * 8, s * 16 * co)


def _mm_body(h_ref, w_ref, o_ref):
    o_ref[...] = jnp.dot(h_ref[...], w_ref[...], preferred_element_type=jnp.float32)


def _mm2_body(h_ref, g_ref, w_ref, wx_ref, o_ref, ox_ref):
    o_ref[...] = jnp.dot(h_ref[...], w_ref[...], preferred_element_type=jnp.float32)
    ox_ref[...] = jnp.dot(g_ref[...], wx_ref[...], preferred_element_type=jnp.float32)


_R = 400


def _transform(h2, wc):
    kk, ko = wc.shape
    return pl.pallas_call(
        _mm_body,
        grid=(_N2 // _R,),
        in_specs=[pl.BlockSpec((_R, kk), lambda i: (i, 0)),
                  pl.BlockSpec((kk, ko), lambda i: (0, 0))],
        out_specs=pl.BlockSpec((_R, ko), lambda i: (i, 0)),
        out_shape=jax.ShapeDtypeStruct((_N2, ko), jnp.float32),
    )(h2, wc)


def _transform2(h2, g2, wc, wxc):
    ko, kx = wc.shape[1], wxc.shape[1]
    kg = wxc.shape[0]
    return pl.pallas_call(
        _mm2_body,
        grid=(_N2 // _R,),
        in_specs=[pl.BlockSpec((_R, 256), lambda i: (i, 0)),
                  pl.BlockSpec((_R, kg), lambda i: (i, 0)),
                  pl.BlockSpec((256, ko), lambda i: (0, 0)),
                  pl.BlockSpec((kg, kx), lambda i: (0, 0))],
        out_specs=[pl.BlockSpec((_R, ko), lambda i: (i, 0)),
                   pl.BlockSpec((_R, kx), lambda i: (i, 0))],
        out_shape=[jax.ShapeDtypeStruct((_N2, ko), jnp.float32),
                   jax.ShapeDtypeStruct((_N2, kx), jnp.float32)],
    )(h2, g2, wc, wxc)


_CH = 5
_SUP = 32
_SPN = _CH * _SUP


def _gather_sum(table, sp, bias, xw=None):
    rw = table.shape[1]
    rc = rw // 16
    gsup_n = _N // _SPN
    nsup = (gsup_n + 31) // 32
    nch = nsup * _SUP
    gl = _CH * 16
    have_xw = xw is not None

    def body(*refs):
        if have_xw:
            (t_h, sp_h, b_h, xw_h, out_h, sp_v, idx_v, g_v, out_v, bias_v,
             xw_v, s_sp0, s_sp1, s_x0, s_x1, s_g0, s_g1, s_o0, s_o1) = refs
        else:
            (t_h, sp_h, b_h, out_h, sp_v, idx_v, g_v, out_v, bias_v,
             s_sp0, s_sp1, s_g0, s_g1, s_o0, s_o1) = refs
            s_x0 = s_x1 = xw_v = None
        wid = lax.axis_index("s") * 2 + lax.axis_index("c")
        pltpu.sync_copy(b_h, bias_v)
        lane2 = lax.iota(jnp.int32, 16) * 2

        def stage_in(si_slot, gsup, sem):
            pltpu.async_copy(sp_h.at[pl.ds(gsup * _SPN, _SPN)],
                             sp_v.at[pl.ds((si_slot % 2) * _SPN, _SPN)], sem)

        def wait_in(sem, dof):
            pltpu.make_async_copy(sp_h.at[pl.ds(0, _SPN)],
                                  sp_v.at[pl.ds(dof, _SPN)], sem).wait()

        def stage_xw(si_slot, gsup, sem):
            pltpu.async_copy(xw_h.at[pl.ds(gsup * _SPN, _SPN)],
                             xw_v.at[pl.ds((si_slot % 2) * _SPN, _SPN)], sem)

        def wait_xw(sem, dof):
            pltpu.make_async_copy(xw_h.at[pl.ds(0, _SPN)],
                                  xw_v.at[pl.ds(dof, _SPN)], sem).wait()

        stage_in(0, wid, s_sp0)
        wait_in(s_sp0, 0)
        stage_in(1, 32 + wid, s_sp1)
        if have_xw:
            stage_xw(0, wid, s_x0)
            wait_xw(s_x0, 0)
            stage_xw(1, 32 + wid, s_x1)

        def step(t, carry):
            si, tin = t // _SUP, t % _SUP
            gb = t % 2
            gsup = si * 32 + wid

            @pl.when((t < nch) & (gsup < gsup_n))
            def _():
                @pl.when((tin == 0) & (si >= 1))
                def _():
                    @pl.when(si % 2 == 0)
                    def _():
                        wait_in(s_sp0, 0)

                    @pl.when(si % 2 == 1)
                    def _():
                        wait_in(s_sp1, _SPN)

                    @pl.when((si + 1 < nsup) & ((si + 1) * 32 + wid < gsup_n))
                    def _():
                        @pl.when((si + 1) % 2 == 0)
                        def _():
                            stage_in(0, (si + 1) * 32 + wid, s_sp0)

                        @pl.when((si + 1) % 2 == 1)
                        def _():
                            stage_in(1, (si + 1) * 32 + wid, s_sp1)

                if have_xw:
                    @pl.when((tin == 1) & (si >= 1) &
                             (si + 1 < nsup) &
                             ((si + 1) * 32 + wid < gsup_n))
                    def _():
                        @pl.when((si + 1) % 2 == 0)
                        def _():
                            stage_xw(0, (si + 1) * 32 + wid, s_x0)

                        @pl.when((si + 1) % 2 == 1)
                        def _():
                            stage_xw(1, (si + 1) * 32 + wid, s_x1)

                srow = (si % 2) * _SPN + tin * _CH
                for j in range(_CH):
                    v = sp_v[srow + j]
                    idx_v[gb, pl.ds(j * 16, 16)] = \
                        ((v >> 1) << 5) + (v & 1) + lane2

                @pl.when(gb == 0)
                def _():
                    pltpu.async_copy(t_h.at[idx_v.at[0]],
                                     g_v.at[pl.ds(0, gl)], s_g0)

                @pl.when(gb == 1)
                def _():
                    pltpu.async_copy(t_h.at[idx_v.at[1]],
                                     g_v.at[pl.ds(gl, gl)], s_g1)

            u = t - 1
            usi, uin = u // _SUP, u % _SUP
            ugsup = usi * 32 + wid

            @pl.when((t >= 1) & (ugsup < gsup_n))
            def _():
                ub = u % 2
                ob = usi % 2

                @pl.when(ub == 0)
                def _():
                    pltpu.make_async_copy(t_h.at[idx_v.at[0]],
                                          g_v.at[pl.ds(0, gl)], s_g0).wait()

                @pl.when(ub == 1)
                def _():
                    pltpu.make_async_copy(t_h.at[idx_v.at[1]],
                                          g_v.at[pl.ds(gl, gl)], s_g1).wait()

                if have_xw:
                    @pl.when((uin == 0) & (usi >= 1))
                    def _():
                        @pl.when(usi % 2 == 0)
                        def _():
                            wait_xw(s_x0, 0)

                        @pl.when(usi % 2 == 1)
                        def _():
                            wait_xw(s_x1, _SPN)

                gof = ub * gl
                orow = ob * _SPN + uin * _CH
                for j in range(_CH):
                    for c in range(rc):
                        sl = pl.ds(c * 16, 16)
                        acc = bias_v[sl]
                        if have_xw:
                            acc = acc + xw_v[orow + j, sl]
                        for k in range(16):
                            acc = acc + g_v[gof + j * 16 + k, sl]
                        out_v[orow + j, sl] = acc

                @pl.when(uin == _SUP - 1)
                def _():
                    dst = ugsup * _SPN

                    @pl.when(ob == 0)
                    def _():
                        pltpu.sync_copy(out_v.at[pl.ds(0, _SPN)],
                                        out_h.at[pl.ds(dst, _SPN)])

                    @pl.when(ob == 1)
                    def _():
                        pltpu.sync_copy(out_v.at[pl.ds(_SPN, _SPN)],
                                        out_h.at[pl.ds(dst, _SPN)])
            return carry

        lax.fori_loop(0, nch + 1, step, 0)

    scratch = [pltpu.VMEM((2 * _SPN, 16), jnp.int32),
               pltpu.VMEM((2, gl), jnp.int32),
               pltpu.VMEM((2 * gl, rw), jnp.float32),
               pltpu.VMEM((2 * _SPN, rw), jnp.float32),
               pltpu.VMEM((rw,), jnp.float32)]
    if have_xw:
        scratch.append(pltpu.VMEM((2 * _SPN, rw), jnp.float32))
    nsem = 8 if have_xw else 6
    scratch += [pltpu.SemaphoreType.DMA] * nsem
    args = (table, sp, bias) + ((xw,) if have_xw else ())
    params = (pltpu.CompilerParams(use_tc_tiling_on_sc=False)
              if rw < 128 else None)
    return pl.kernel(body,
                     out_type=jax.ShapeDtypeStruct((_N, rw), jnp.float32),
                     mesh=_SC_MESH,
                     compiler_params=params,
                     scratch_types=scratch)(*args)


def kernel(x, spiral, W0, b0, W1, b1, W2, b2, W_out, b_out):
    sp = spiral.astype(jnp.int32)

    w0b = W0.reshape(_S, 3, 16)
    w1b = W1.reshape(_S, 16, 16)
    wh, wx = W_out[:16, :], W_out[16:, :]
    w2f = (W2 @ wh).reshape(_S, 16, 3)
    w2b = jnp.pad(w2f, ((0, 0), (0, 0), (0, 1)))
    wc0 = _make_wcat_cb(w0b, 16)
    wc1, wc2 = _make_wcat(w1b), _make_wcat(w2b)
    wxp = jnp.pad(wx, ((0, 0), (0, 1)))
    wxc = _make_wcat_cb(wxp[None], 4)
    bias0 = jnp.tile(b0, (_B,))
    bias1 = jnp.tile(b1, (_B,))
    bias2 = jnp.tile(jnp.pad(b2 @ wh + b_out, (0, 1)), (_B,))

    xt = jnp.transpose(x.reshape(_B, _N * 3)).reshape(_N2, 48)

    t0 = _transform(xt, wc0).reshape(_N2 * 32, 128)
    h1 = _gather_sum(t0, sp, bias0)
    t1 = _transform(h1.reshape(_N2, 256), wc1).reshape(_N2 * 32, 128)
    h2 = _gather_sum(t1, sp, bias1)
    t2, xw = _transform2(h2.reshape(_N2, 256), xt, wc2, wxc)
    outp = _gather_sum(t2.reshape(_N2 * 32, 32), sp, bias2,
                       xw=xw.reshape(_N, 32))

    return jnp.transpose(outp.reshape(_N, _B, 4)[:, :, :3], (1, 0, 2))


# speedup vs baseline: 31.8968x; 1.0301x over previous
import jax
import jax.numpy as jnp
from jax import lax
from jax.experimental import pallas as pl
from jax.experimental.pallas import tpu as pltpu
from jax.experimental.pallas import tpu_sc as plsc

_N = 100000
_S = 16
_B = 8
_N2 = _N // 2

_SC_MESH = plsc.VectorSubcoreMesh(core_axis_name="c", subcore_axis_name="s")


def _make_wcat(Wb):
    s, _, co = Wb.shape
    eye = jnp.eye(16, dtype=Wb.dtype)
    k = eye[None, :, None, :, None] * Wb[:, None, :, None, :]
    return k.reshape(s, 256, 16 * co).transpose(1, 0, 2).reshape(256, s * 16 * co)


def _make_wcat_cb(Wb, co_out):
    s, ci, co = Wb.shape
    e8 = jnp.eye(8, dtype=Wb.dtype)
    k = Wb[:, :, None, None, :] * e8[None, None, :, :, None]
    k = k.reshape(s, ci * 8, 8 * co)
    e2 = jnp.eye(2, dtype=Wb.dtype)
    k2 = (e2[None, :, None, :, None] * k[:, None, :, None, :])
    k2 = k2.reshape(s, 2 * ci * 8, 2 * 8 * co)
    assert co == co_out
    return k2.transpose(1, 0, 2).reshape(2 * ci * 8, s * 16 * co)


def _round_bf16_bits(f):
    b = lax.bitcast_convert_type(f, jnp.int32)
    return ((b + 0x7FFF + ((b >> 16) & 1)) >> 16) & 0xFFFF


def _mm_s_body(h_ref, w_ref, o_ref):
    v = jnp.dot(h_ref[...].astype(w_ref.dtype), w_ref[0],
                preferred_element_type=jnp.float32)
    for c in range(4):
        lo = _round_bf16_bits(v[:, c * 32:c * 32 + 16])
        hi = _round_bf16_bits(v[:, c * 32 + 16:c * 32 + 32])
        o_ref[0, :, c * 16:(c + 1) * 16] = lo | (hi << 16)


def _mm_body(h_ref, w_ref, o_ref):
    o_ref[...] = jnp.dot(h_ref[...], w_ref[...], preferred_element_type=jnp.float32)


def _mm2_body(h_ref, g_ref, w_ref, wx_ref, o_ref, ox_ref):
    o_ref[...] = jnp.dot(h_ref[...], w_ref[...], preferred_element_type=jnp.float32)
    ox_ref[...] = jnp.dot(g_ref[...], wx_ref[...], preferred_element_type=jnp.float32)


_R = 400
_RN = 1000


def _make_ws(Wb):
    s = Wb.shape[0]
    e8 = jnp.eye(8, dtype=Wb.dtype)
    k = e8[None, :, None, :, None] * Wb[:, None, :, None, :]
    return k.reshape(s, 128, 128)


def _make_ws_cb(Wb):
    s, ci, co = Wb.shape
    e8 = jnp.eye(8, dtype=Wb.dtype)
    k = Wb[:, :, None, None, :] * e8[None, None, :, :, None]
    return k.reshape(s, ci * 8, 8 * co)


def _transform_s(h, w_all):
    kk = w_all.shape[1]
    return pl.pallas_call(
        _mm_s_body,
        grid=(_N // _RN, 16),
        in_specs=[pl.BlockSpec((_RN, kk), lambda i, s: (i, 0)),
                  pl.BlockSpec((1, kk, 128), lambda i, s: (s, 0, 0))],
        out_specs=pl.BlockSpec((1, _RN, 64), lambda i, s: (s, i, 0)),
        out_shape=jax.ShapeDtypeStruct((16, _N, 64), jnp.int32),
    )(h, w_all)


def _transform(h2, wc):
    kk, ko = wc.shape
    return pl.pallas_call(
        _mm_body,
        grid=(_N2 // _R,),
        in_specs=[pl.BlockSpec((_R, kk), lambda i: (i, 0)),
                  pl.BlockSpec((kk, ko), lambda i: (0, 0))],
        out_specs=pl.BlockSpec((_R, ko), lambda i: (i, 0)),
        out_shape=jax.ShapeDtypeStruct((_N2, ko), jnp.float32),
    )(h2, wc)


def _transform2(h2, g2, wc, wxc):
    ko, kx = wc.shape[1], wxc.shape[1]
    kg = wxc.shape[0]
    return pl.pallas_call(
        _mm2_body,
        grid=(_N2 // _R,),
        in_specs=[pl.BlockSpec((_R, 256), lambda i: (i, 0)),
                  pl.BlockSpec((_R, kg), lambda i: (i, 0)),
                  pl.BlockSpec((256, ko), lambda i: (0, 0)),
                  pl.BlockSpec((kg, kx), lambda i: (0, 0))],
        out_specs=[pl.BlockSpec((_R, ko), lambda i: (i, 0)),
                   pl.BlockSpec((_R, kx), lambda i: (i, 0))],
        out_shape=[jax.ShapeDtypeStruct((_N2, ko), jnp.float32),
                   jax.ShapeDtypeStruct((_N2, kx), jnp.float32)],
    )(h2, g2, wc, wxc)


_CH = 5
_SUP = 32
_SPN = _CH * _SUP


def _gather_sum(table, sp, bias, xw=None, packed=False):
    rw = table.shape[1]
    ow = 2 * rw if packed else rw
    rc = rw // 16
    gsup_n = _N // _SPN
    nsup = (gsup_n + 31) // 32
    nch = nsup * _SUP
    gl = _CH * 16
    have_xw = xw is not None

    def body(*refs):
        if have_xw:
            (t_h, sp_h, b_h, xw_h, out_h, sp_v, idx_v, g_v, out_v, bias_v,
             xw_v, s_sp0, s_sp1, s_x0, s_x1, s_g0, s_g1, s_o0, s_o1) = refs
        else:
            (t_h, sp_h, b_h, out_h, sp_v, idx_v, g_v, out_v, bias_v,
             s_sp0, s_sp1, s_g0, s_g1, s_o0, s_o1) = refs
            s_x0 = s_x1 = xw_v = None
        wid = lax.axis_index("s") * 2 + lax.axis_index("c")
        pltpu.sync_copy(b_h, bias_v)
        lane2 = (lax.iota(jnp.int32, 16) * _N if packed
                 else lax.iota(jnp.int32, 16) * 2)

        def stage_in(si_slot, gsup, sem):
            pltpu.async_copy(sp_h.at[pl.ds(gsup * _SPN, _SPN)],
                             sp_v.at[pl.ds((si_slot % 2) * _SPN, _SPN)], sem)

        def wait_in(sem, dof):
            pltpu.make_async_copy(sp_h.at[pl.ds(0, _SPN)],
                                  sp_v.at[pl.ds(dof, _SPN)], sem).wait()

        def stage_xw(si_slot, gsup, sem):
            pltpu.async_copy(xw_h.at[pl.ds(gsup * _SPN, _SPN)],
                             xw_v.at[pl.ds((si_slot % 2) * _SPN, _SPN)], sem)

        def wait_xw(sem, dof):
            pltpu.make_async_copy(xw_h.at[pl.ds(0, _SPN)],
                                  xw_v.at[pl.ds(dof, _SPN)], sem).wait()

        stage_in(0, wid, s_sp0)
        wait_in(s_sp0, 0)
        stage_in(1, 32 + wid, s_sp1)
        if have_xw:
            stage_xw(0, wid, s_x0)
            wait_xw(s_x0, 0)
            stage_xw(1, 32 + wid, s_x1)

        def step(t, carry):
            si, tin = t // _SUP, t % _SUP
            gb = t % 2
            gsup = si * 32 + wid

            @pl.when((t < nch) & (gsup < gsup_n))
            def _():
                @pl.when((tin == 0) & (si >= 1))
                def _():
                    @pl.when(si % 2 == 0)
                    def _():
                        wait_in(s_sp0, 0)

                    @pl.when(si % 2 == 1)
                    def _():
                        wait_in(s_sp1, _SPN)

                    @pl.when((si + 1 < nsup) & ((si + 1) * 32 + wid < gsup_n))
                    def _():
                        @pl.when((si + 1) % 2 == 0)
                        def _():
                            stage_in(0, (si + 1) * 32 + wid, s_sp0)

                        @pl.when((si + 1) % 2 == 1)
                        def _():
                            stage_in(1, (si + 1) * 32 + wid, s_sp1)

                if have_xw:
                    @pl.when((tin == 1) & (si >= 1) &
                             (si + 1 < nsup) &
                             ((si + 1) * 32 + wid < gsup_n))
                    def _():
                        @pl.when((si + 1) % 2 == 0)
                        def _():
                            stage_xw(0, (si + 1) * 32 + wid, s_x0)

                        @pl.when((si + 1) % 2 == 1)
                        def _():
                            stage_xw(1, (si + 1) * 32 + wid, s_x1)

                srow = (si % 2) * _SPN + tin * _CH
                for j in range(_CH):
                    v = sp_v[srow + j]
                    if packed:
                        idx_v[gb, pl.ds(j * 16, 16)] = v + lane2
                    else:
                        idx_v[gb, pl.ds(j * 16, 16)] = \
                            ((v >> 1) << 5) + (v & 1) + lane2

                @pl.when(gb == 0)
                def _():
                    pltpu.async_copy(t_h.at[idx_v.at[0]],
                                     g_v.at[pl.ds(0, gl)], s_g0)

                @pl.when(gb == 1)
                def _():
                    pltpu.async_copy(t_h.at[idx_v.at[1]],
                                     g_v.at[pl.ds(gl, gl)], s_g1)

            u = t - 1
            usi, uin = u // _SUP, u % _SUP
            ugsup = usi * 32 + wid

            @pl.when((t >= 1) & (ugsup < gsup_n))
            def _():
                ub = u % 2
                ob = usi % 2

                @pl.when(ub == 0)
                def _():
                    pltpu.make_async_copy(t_h.at[idx_v.at[0]],
                                          g_v.at[pl.ds(0, gl)], s_g0).wait()

                @pl.when(ub == 1)
                def _():
                    pltpu.make_async_copy(t_h.at[idx_v.at[1]],
                                          g_v.at[pl.ds(gl, gl)], s_g1).wait()

                if have_xw:
                    @pl.when((uin == 0) & (usi >= 1))
                    def _():
                        @pl.when(usi % 2 == 0)
                        def _():
                            wait_xw(s_x0, 0)

                        @pl.when(usi % 2 == 1)
                        def _():
                            wait_xw(s_x1, _SPN)

                gof = ub * gl
                orow = ob * _SPN + uin * _CH
                for j in range(_CH):
                    for c in range(rc):
                        sl = pl.ds(c * 16, 16)
                        if packed:
                            lo = bias_v[pl.ds(c * 32, 16)]
                            hi = bias_v[pl.ds(c * 32 + 16, 16)]
                            for k in range(16):
                                w = g_v[gof + j * 16 + k, sl]
                                lo = lo + lax.bitcast_convert_type(
                                    w << 16, jnp.float32)
                                hi = hi + lax.bitcast_convert_type(
                                    w & jnp.int32(-65536), jnp.float32)
                            out_v[orow + j, pl.ds(c * 32, 16)] = lo
                            out_v[orow + j, pl.ds(c * 32 + 16, 16)] = hi
                        else:
                            acc = bias_v[sl]
                            if have_xw:
                                acc = acc + xw_v[orow + j, sl]
                            for k in range(16):
                                acc = acc + g_v[gof + j * 16 + k, sl]
                            out_v[orow + j, sl] = acc

                @pl.when(uin == _SUP - 1)
                def _():
                    dst = ugsup * _SPN

                    @pl.when(ob == 0)
                    def _():
                        pltpu.sync_copy(out_v.at[pl.ds(0, _SPN)],
                                        out_h.at[pl.ds(dst, _SPN)])

                    @pl.when(ob == 1)
                    def _():
                        pltpu.sync_copy(out_v.at[pl.ds(_SPN, _SPN)],
                                        out_h.at[pl.ds(dst, _SPN)])
            return carry

        lax.fori_loop(0, nch + 1, step, 0)

    scratch = [pltpu.VMEM((2 * _SPN, 16), jnp.int32),
               pltpu.VMEM((2, gl), jnp.int32),
               pltpu.VMEM((2 * gl, rw), jnp.int32 if packed else jnp.float32),
               pltpu.VMEM((2 * _SPN, ow), jnp.float32),
               pltpu.VMEM((ow,), jnp.float32)]
    if have_xw:
        scratch.append(pltpu.VMEM((2 * _SPN, rw), jnp.float32))
    nsem = 8 if have_xw else 6
    scratch += [pltpu.SemaphoreType.DMA] * nsem
    args = (table, sp, bias) + ((xw,) if have_xw else ())
    params = (pltpu.CompilerParams(use_tc_tiling_on_sc=False)
              if rw < 128 else None)
    return pl.kernel(body,
                     out_type=jax.ShapeDtypeStruct((_N, ow), jnp.float32),
                     mesh=_SC_MESH,
                     compiler_params=params,
                     scratch_types=scratch)(*args)


def kernel(x, spiral, W0, b0, W1, b1, W2, b2, W_out, b_out):
    sp = spiral.astype(jnp.int32)

    w0b = W0.reshape(_S, 3, 16)
    w1b = W1.reshape(_S, 16, 16)
    wh, wx = W_out[:16, :], W_out[16:, :]
    w2f = (W2 @ wh).reshape(_S, 16, 3)
    w2b = jnp.pad(w2f, ((0, 0), (0, 0), (0, 1)))
    w0s = _make_ws_cb(w0b).astype(jnp.bfloat16)
    w1s = _make_ws(w1b).astype(jnp.bfloat16)
    wc2 = _make_wcat(w2b)
    wxp = jnp.pad(wx, ((0, 0), (0, 1)))
    wxc = _make_wcat_cb(wxp[None], 4)
    bias0 = jnp.tile(b0, (_B,))
    bias1 = jnp.tile(b1, (_B,))
    bias2 = jnp.tile(jnp.pad(b2 @ wh + b_out, (0, 1)), (_B,))

    xt = jnp.transpose(x.reshape(_B, _N * 3))

    t0 = _transform_s(xt.reshape(_N, 24), w0s).reshape(16 * _N, 64)
    h1 = _gather_sum(t0, sp, bias0, packed=True)
    t1 = _transform_s(h1, w1s).reshape(16 * _N, 64)
    h2 = _gather_sum(t1, sp, bias1, packed=True)
    t2, xw = _transform2(h2.reshape(_N2, 256), xt.reshape(_N2, 48), wc2, wxc)
    outp = _gather_sum(t2.reshape(_N2 * 32, 32), sp, bias2,
                       xw=xw.reshape(_N, 32))

    return jnp.transpose(outp.reshape(_N, _B, 4)[:, :, :3], (1, 0, 2))


# speedup vs baseline: 61.5488x; 1.9296x over previous
import jax
import jax.numpy as jnp
from jax import lax
from jax.experimental import pallas as pl
from jax.experimental.pallas import tpu as pltpu
from jax.experimental.pallas import tpu_sc as plsc

_N = 100000
_S = 16
_B = 8
_N2 = _N // 2

_SC_MESH = plsc.VectorSubcoreMesh(core_axis_name="c", subcore_axis_name="s")


def _make_wcat(Wb):
    s, _, co = Wb.shape
    eye = jnp.eye(16, dtype=Wb.dtype)
    k = eye[None, :, None, :, None] * Wb[:, None, :, None, :]
    return k.reshape(s, 256, 16 * co).transpose(1, 0, 2).reshape(256, s * 16 * co)


def _make_wcat_cb(Wb, co_out):
    s, ci, co = Wb.shape
    e8 = jnp.eye(8, dtype=Wb.dtype)
    k = Wb[:, :, None, None, :] * e8[None, None, :, :, None]
    k = k.reshape(s, ci * 8, 8 * co)
    e2 = jnp.eye(2, dtype=Wb.dtype)
    k2 = (e2[None, :, None, :, None] * k[:, None, :, None, :])
    k2 = k2.reshape(s, 2 * ci * 8, 2 * 8 * co)
    assert co == co_out
    return k2.transpose(1, 0, 2).reshape(2 * ci * 8, s * 16 * co)


def _mm_p_body(h_ref, w_ref, o_ref):
    v = jnp.dot(h_ref[...].astype(w_ref.dtype), w_ref[...],
                preferred_element_type=jnp.float32)
    b = lax.bitcast_convert_type(v, jnp.int32)
    n = b.shape[1] // 2
    o_ref[...] = (((b[:, :n] >> 16) & 0xFFFF)
                  | (b[:, n:] & jnp.int32(-65536)))


def _mm_body(h_ref, w_ref, o_ref):
    o_ref[...] = jnp.dot(h_ref[...], w_ref[...], preferred_element_type=jnp.float32)


def _mm2_body(h_ref, g_ref, w_ref, wx_ref, o_ref, ox_ref):
    o_ref[...] = jnp.dot(h_ref[...], w_ref[...], preferred_element_type=jnp.float32)
    ox_ref[...] = jnp.dot(g_ref[...], wx_ref[...], preferred_element_type=jnp.float32)


_R = 400
def _halves_perm(ko):
    import numpy as _np
    m = _np.arange(ko // 2)
    s, d, q = m // 128, (m % 128) // 64, m % 64
    lo = s * 256 + d * 128 + q
    return _np.concatenate([lo, lo + 64])


def _transform_p(h2, wc):
    kk, ko = wc.shape
    return pl.pallas_call(
        _mm_p_body,
        grid=(_N2 // _R,),
        in_specs=[pl.BlockSpec((_R, kk), lambda i: (i, 0)),
                  pl.BlockSpec((kk, ko), lambda i: (0, 0))],
        out_specs=pl.BlockSpec((_R, ko // 2), lambda i: (i, 0)),
        out_shape=jax.ShapeDtypeStruct((_N2, ko // 2), jnp.int32),
    )(h2, wc)


def _transform(h2, wc):
    kk, ko = wc.shape
    return pl.pallas_call(
        _mm_body,
        grid=(_N2 // _R,),
        in_specs=[pl.BlockSpec((_R, kk), lambda i: (i, 0)),
                  pl.BlockSpec((kk, ko), lambda i: (0, 0))],
        out_specs=pl.BlockSpec((_R, ko), lambda i: (i, 0)),
        out_shape=jax.ShapeDtypeStruct((_N2, ko), jnp.float32),
    )(h2, wc)


def _transform2(h2, g2, wc, wxc):
    ko, kx = wc.shape[1], wxc.shape[1]
    kg = wxc.shape[0]
    return pl.pallas_call(
        _mm2_body,
        grid=(_N2 // _R,),
        in_specs=[pl.BlockSpec((_R, 256), lambda i: (i, 0)),
                  pl.BlockSpec((_R, kg), lambda i: (i, 0)),
                  pl.BlockSpec((256, ko), lambda i: (0, 0)),
                  pl.BlockSpec((kg, kx), lambda i: (0, 0))],
        out_specs=[pl.BlockSpec((_R, ko), lambda i: (i, 0)),
                   pl.BlockSpec((_R, kx), lambda i: (i, 0))],
        out_shape=[jax.ShapeDtypeStruct((_N2, ko), jnp.float32),
                   jax.ShapeDtypeStruct((_N2, kx), jnp.float32)],
    )(h2, g2, wc, wxc)


_CH = 5
_SUP = 32
_SPN = _CH * _SUP


def _gather_sum(table, sp, bias, xw=None, packed=False):
    rw = table.shape[1]
    ow = 2 * rw if packed else rw
    rc = rw // 16
    gsup_n = _N // _SPN
    nsup = (gsup_n + 31) // 32
    nch = nsup * _SUP
    gl = _CH * 16
    have_xw = xw is not None

    def body(*refs):
        if have_xw:
            (t_h, sp_h, b_h, xw_h, out_h, sp_v, idx_v, g_v, out_v, bias_v,
             xw_v, s_sp0, s_sp1, s_x0, s_x1, s_g0, s_g1, s_o0, s_o1) = refs
        else:
            (t_h, sp_h, b_h, out_h, sp_v, idx_v, g_v, out_v, bias_v,
             s_sp0, s_sp1, s_g0, s_g1, s_o0, s_o1) = refs
            s_x0 = s_x1 = xw_v = None
        wid = lax.axis_index("s") * 2 + lax.axis_index("c")
        pltpu.sync_copy(b_h, bias_v)
        lane2 = lax.iota(jnp.int32, 16) * 2

        def stage_in(si_slot, gsup, sem):
            pltpu.async_copy(sp_h.at[pl.ds(gsup * _SPN, _SPN)],
                             sp_v.at[pl.ds((si_slot % 2) * _SPN, _SPN)], sem)

        def wait_in(sem, dof):
            pltpu.make_async_copy(sp_h.at[pl.ds(0, _SPN)],
                                  sp_v.at[pl.ds(dof, _SPN)], sem).wait()

        def stage_xw(si_slot, gsup, sem):
            pltpu.async_copy(xw_h.at[pl.ds(gsup * _SPN, _SPN)],
                             xw_v.at[pl.ds((si_slot % 2) * _SPN, _SPN)], sem)

        def wait_xw(sem, dof):
            pltpu.make_async_copy(xw_h.at[pl.ds(0, _SPN)],
                                  xw_v.at[pl.ds(dof, _SPN)], sem).wait()

        stage_in(0, wid, s_sp0)
        wait_in(s_sp0, 0)
        stage_in(1, 32 + wid, s_sp1)
        if have_xw:
            stage_xw(0, wid, s_x0)
            wait_xw(s_x0, 0)
            stage_xw(1, 32 + wid, s_x1)

        def step(t, carry):
            si, tin = t // _SUP, t % _SUP
            gb = t % 2
            gsup = si * 32 + wid

            @pl.when((t < nch) & (gsup < gsup_n))
            def _():
                @pl.when((tin == 0) & (si >= 1))
                def _():
                    @pl.when(si % 2 == 0)
                    def _():
                        wait_in(s_sp0, 0)

                    @pl.when(si % 2 == 1)
                    def _():
                        wait_in(s_sp1, _SPN)

                    @pl.when((si + 1 < nsup) & ((si + 1) * 32 + wid < gsup_n))
                    def _():
                        @pl.when((si + 1) % 2 == 0)
                        def _():
                            stage_in(0, (si + 1) * 32 + wid, s_sp0)

                        @pl.when((si + 1) % 2 == 1)
                        def _():
                            stage_in(1, (si + 1) * 32 + wid, s_sp1)

                if have_xw:
                    @pl.when((tin == 1) & (si >= 1) &
                             (si + 1 < nsup) &
                             ((si + 1) * 32 + wid < gsup_n))
                    def _():
                        @pl.when((si + 1) % 2 == 0)
                        def _():
                            stage_xw(0, (si + 1) * 32 + wid, s_x0)

                        @pl.when((si + 1) % 2 == 1)
                        def _():
                            stage_xw(1, (si + 1) * 32 + wid, s_x1)

                srow = (si % 2) * _SPN + tin * _CH
                for j in range(_CH):
                    v = sp_v[srow + j]
                    idx_v[gb, pl.ds(j * 16, 16)] = \
                        ((v >> 1) << 5) + (v & 1) + lane2

                @pl.when(gb == 0)
                def _():
                    pltpu.async_copy(t_h.at[idx_v.at[0]],
                                     g_v.at[pl.ds(0, gl)], s_g0)

                @pl.when(gb == 1)
                def _():
                    pltpu.async_copy(t_h.at[idx_v.at[1]],
                                     g_v.at[pl.ds(gl, gl)], s_g1)

            u = t - 1
            usi, uin = u // _SUP, u % _SUP
            ugsup = usi * 32 + wid

            @pl.when((t >= 1) & (ugsup < gsup_n))
            def _():
                ub = u % 2
                ob = usi % 2

                @pl.when(ub == 0)
                def _():
                    pltpu.make_async_copy(t_h.at[idx_v.at[0]],
                                          g_v.at[pl.ds(0, gl)], s_g0).wait()

                @pl.when(ub == 1)
                def _():
                    pltpu.make_async_copy(t_h.at[idx_v.at[1]],
                                          g_v.at[pl.ds(gl, gl)], s_g1).wait()

                if have_xw:
                    @pl.when((uin == 0) & (usi >= 1))
                    def _():
                        @pl.when(usi % 2 == 0)
                        def _():
                            wait_xw(s_x0, 0)

                        @pl.when(usi % 2 == 1)
                        def _():
                            wait_xw(s_x1, _SPN)

                gof = ub * gl
                orow = ob * _SPN + uin * _CH
                for j in range(_CH):
                    for c in range(rc):
                        sl = pl.ds(c * 16, 16)
                        if packed:
                            lo = bias_v[pl.ds(c * 16, 16)]
                            hi = bias_v[pl.ds(64 + c * 16, 16)]
                            for k in range(16):
                                w = g_v[gof + j * 16 + k, sl]
                                lo = lo + lax.bitcast_convert_type(
                                    w << 16, jnp.float32)
                                hi = hi + lax.bitcast_convert_type(
                                    w & jnp.int32(-65536), jnp.float32)
                            out_v[orow + j, pl.ds(c * 16, 16)] = lo
                            out_v[orow + j, pl.ds(64 + c * 16, 16)] = hi
                        else:
                            acc = bias_v[sl]
                            if have_xw:
                                acc = acc + xw_v[orow + j, sl]
                            for k in range(16):
                                acc = acc + g_v[gof + j * 16 + k, sl]
                            out_v[orow + j, sl] = acc

                @pl.when(uin == _SUP - 1)
                def _():
                    dst = ugsup * _SPN

                    @pl.when(ob == 0)
                    def _():
                        pltpu.sync_copy(out_v.at[pl.ds(0, _SPN)],
                                        out_h.at[pl.ds(dst, _SPN)])

                    @pl.when(ob == 1)
                    def _():
                        pltpu.sync_copy(out_v.at[pl.ds(_SPN, _SPN)],
                                        out_h.at[pl.ds(dst, _SPN)])
            return carry

        lax.fori_loop(0, nch + 1, step, 0)

    scratch = [pltpu.VMEM((2 * _SPN, 16), jnp.int32),
               pltpu.VMEM((2, gl), jnp.int32),
               pltpu.VMEM((2 * gl, rw), jnp.int32 if packed else jnp.float32),
               pltpu.VMEM((2 * _SPN, ow), jnp.float32),
               pltpu.VMEM((ow,), jnp.float32)]
    if have_xw:
        scratch.append(pltpu.VMEM((2 * _SPN, rw), jnp.float32))
    nsem = 8 if have_xw else 6
    scratch += [pltpu.SemaphoreType.DMA] * nsem
    args = (table, sp, bias) + ((xw,) if have_xw else ())
    params = (pltpu.CompilerParams(use_tc_tiling_on_sc=False)
              if rw < 128 else None)
    return pl.kernel(body,
                     out_type=jax.ShapeDtypeStruct((_N, ow), jnp.float32),
                     mesh=_SC_MESH,
                     compiler_params=params,
                     scratch_types=scratch)(*args)


def kernel(x, spiral, W0, b0, W1, b1, W2, b2, W_out, b_out):
    sp = spiral.astype(jnp.int32)

    w0b = W0.reshape(_S, 3, 16)
    w1b = W1.reshape(_S, 16, 16)
    wh, wx = W_out[:16, :], W_out[16:, :]
    w2f = (W2 @ wh).reshape(_S, 16, 3)
    w2b = jnp.pad(w2f, ((0, 0), (0, 0), (0, 1)))
    hp = _halves_perm(4096)
    wc0 = _make_wcat_cb(w0b, 16)[:, hp].astype(jnp.bfloat16)
    wc1 = _make_wcat(w1b)[:, hp].astype(jnp.bfloat16)
    wc2 = _make_wcat(w2b)
    wxp = jnp.pad(wx, ((0, 0), (0, 1)))
    wxc = _make_wcat_cb(wxp[None], 4)
    bias0 = jnp.tile(b0, (_B,))
    bias1 = jnp.tile(b1, (_B,))
    bias2 = jnp.tile(jnp.pad(b2 @ wh + b_out, (0, 1)), (_B,))

    xt = jnp.transpose(x.reshape(_B, _N * 3))

    t0 = _transform_p(xt.reshape(_N2, 48), wc0).reshape(_N2 * 32, 64)
    h1 = _gather_sum(t0, sp, bias0, packed=True)
    t1 = _transform_p(h1.reshape(_N2, 256), wc1).reshape(_N2 * 32, 64)
    h2 = _gather_sum(t1, sp, bias1, packed=True)
    t2, xw = _transform2(h2.reshape(_N2, 256), xt.reshape(_N2, 48), wc2, wxc)
    outp = _gather_sum(t2.reshape(_N2 * 32, 32), sp, bias2,
                       xw=xw.reshape(_N, 32))

    return jnp.transpose(outp.reshape(_N, _B, 4)[:, :, :3], (1, 0, 2))


# speedup vs baseline: 66.0775x; 1.0736x over previous
import jax
import jax.numpy as jnp
from jax import lax
from jax.experimental import pallas as pl
from jax.experimental.pallas import tpu as pltpu
from jax.experimental.pallas import tpu_sc as plsc

_N = 100000
_S = 16
_B = 8
_N2 = _N // 2

_SC_MESH = plsc.VectorSubcoreMesh(core_axis_name="c", subcore_axis_name="s")


def _make_wcat(Wb):
    s, _, co = Wb.shape
    eye = jnp.eye(16, dtype=Wb.dtype)
    k = eye[None, :, None, :, None] * Wb[:, None, :, None, :]
    return k.reshape(s, 256, 16 * co).transpose(1, 0, 2).reshape(256, s * 16 * co)


def _make_wcat_cb(Wb, co_out):
    s, ci, co = Wb.shape
    e8 = jnp.eye(8, dtype=Wb.dtype)
    k = Wb[:, :, None, None, :] * e8[None, None, :, :, None]
    k = k.reshape(s, ci * 8, 8 * co)
    e2 = jnp.eye(2, dtype=Wb.dtype)
    k2 = (e2[None, :, None, :, None] * k[:, None, :, None, :])
    k2 = k2.reshape(s, 2 * ci * 8, 2 * 8 * co)
    assert co == co_out
    return k2.transpose(1, 0, 2).reshape(2 * ci * 8, s * 16 * co)


def _mm_p_body(h_ref, w_ref, o_ref):
    v = jnp.dot(h_ref[...].astype(w_ref.dtype), w_ref[...],
                preferred_element_type=jnp.float32)
    b = lax.bitcast_convert_type(v, jnp.int32)
    n = b.shape[1] // 2
    o_ref[...] = (((b[:, :n] >> 16) & 0xFFFF)
                  | (b[:, n:] & jnp.int32(-65536)))


def _mm_body(h_ref, w_ref, o_ref):
    o_ref[...] = jnp.dot(h_ref[...], w_ref[...], preferred_element_type=jnp.float32)


def _mm2_body(h_ref, g_ref, w_ref, wx_ref, o_ref, ox_ref):
    o_ref[...] = jnp.dot(h_ref[...], w_ref[...], preferred_element_type=jnp.float32)
    ox_ref[...] = jnp.dot(g_ref[...], wx_ref[...], preferred_element_type=jnp.float32)


_R = 400
def _halves_perm(ko):
    import numpy as _np
    m = _np.arange(ko // 2)
    s, d, q = m // 128, (m % 128) // 64, m % 64
    lo = s * 256 + d * 128 + q
    return _np.concatenate([lo, lo + 64])


def _transform_p(h2, wc):
    kk, ko = wc.shape
    return pl.pallas_call(
        _mm_p_body,
        grid=(_N2 // _R,),
        in_specs=[pl.BlockSpec((_R, kk), lambda i: (i, 0)),
                  pl.BlockSpec((kk, ko), lambda i: (0, 0))],
        out_specs=pl.BlockSpec((_R, ko // 2), lambda i: (i, 0)),
        out_shape=jax.ShapeDtypeStruct((_N2, ko // 2), jnp.int32),
    )(h2, wc)


def _transform(h2, wc):
    kk, ko = wc.shape
    return pl.pallas_call(
        _mm_body,
        grid=(_N2 // _R,),
        in_specs=[pl.BlockSpec((_R, kk), lambda i: (i, 0)),
                  pl.BlockSpec((kk, ko), lambda i: (0, 0))],
        out_specs=pl.BlockSpec((_R, ko), lambda i: (i, 0)),
        out_shape=jax.ShapeDtypeStruct((_N2, ko), jnp.float32),
    )(h2, wc)


def _transform2(h2, g2, wc, wxc):
    ko, kx = wc.shape[1], wxc.shape[1]
    kg = wxc.shape[0]
    return pl.pallas_call(
        _mm2_body,
        grid=(_N2 // _R,),
        in_specs=[pl.BlockSpec((_R, 256), lambda i: (i, 0)),
                  pl.BlockSpec((_R, kg), lambda i: (i, 0)),
                  pl.BlockSpec((256, ko), lambda i: (0, 0)),
                  pl.BlockSpec((kg, kx), lambda i: (0, 0))],
        out_specs=[pl.BlockSpec((_R, ko), lambda i: (i, 0)),
                   pl.BlockSpec((_R, kx), lambda i: (i, 0))],
        out_shape=[jax.ShapeDtypeStruct((_N2, ko), jnp.float32),
                   jax.ShapeDtypeStruct((_N2, kx), jnp.float32)],
    )(h2, g2, wc, wxc)


_CH = 8
_SUP = 20
_SPN = _CH * _SUP


def _gather_sum(table, sp, bias, xw=None, packed=False):
    rw = table.shape[1]
    ow = 2 * rw if packed else rw
    rc = rw // 16
    gsup_n = _N // _SPN
    nsup = (gsup_n + 31) // 32
    nch = nsup * _SUP
    gl = _CH * 16
    have_xw = xw is not None

    def body(*refs):
        if have_xw:
            (t_h, sp_h, b_h, xw_h, out_h, sp_v, idx_v, g_v, out_v, bias_v,
             xw_v, s_sp0, s_sp1, s_x0, s_x1, s_g0, s_g1, s_o0, s_o1) = refs
        else:
            (t_h, sp_h, b_h, out_h, sp_v, idx_v, g_v, out_v, bias_v,
             s_sp0, s_sp1, s_g0, s_g1, s_o0, s_o1) = refs
            s_x0 = s_x1 = xw_v = None
        wid = lax.axis_index("s") * 2 + lax.axis_index("c")
        pltpu.sync_copy(b_h, bias_v)
        lane2 = lax.iota(jnp.int32, 16) * 2

        def stage_in(si_slot, gsup, sem):
            pltpu.async_copy(sp_h.at[pl.ds(gsup * _SPN, _SPN)],
                             sp_v.at[pl.ds((si_slot % 2) * _SPN, _SPN)], sem)

        def wait_in(sem, dof):
            pltpu.make_async_copy(sp_h.at[pl.ds(0, _SPN)],
                                  sp_v.at[pl.ds(dof, _SPN)], sem).wait()

        def stage_xw(si_slot, gsup, sem):
            pltpu.async_copy(xw_h.at[pl.ds(gsup * _SPN, _SPN)],
                             xw_v.at[pl.ds((si_slot % 2) * _SPN, _SPN)], sem)

        def wait_xw(sem, dof):
            pltpu.make_async_copy(xw_h.at[pl.ds(0, _SPN)],
                                  xw_v.at[pl.ds(dof, _SPN)], sem).wait()

        stage_in(0, wid, s_sp0)
        wait_in(s_sp0, 0)
        stage_in(1, 32 + wid, s_sp1)
        if have_xw:
            stage_xw(0, wid, s_x0)
            wait_xw(s_x0, 0)
            stage_xw(1, 32 + wid, s_x1)

        def step(t, carry):
            si, tin = t // _SUP, t % _SUP
            gb = t % 2
            gsup = si * 32 + wid

            @pl.when((t < nch) & (gsup < gsup_n))
            def _():
                @pl.when((tin == 0) & (si >= 1))
                def _():
                    @pl.when(si % 2 == 0)
                    def _():
                        wait_in(s_sp0, 0)

                    @pl.when(si % 2 == 1)
                    def _():
                        wait_in(s_sp1, _SPN)

                    @pl.when((si + 1 < nsup) & ((si + 1) * 32 + wid < gsup_n))
                    def _():
                        @pl.when((si + 1) % 2 == 0)
                        def _():
                            stage_in(0, (si + 1) * 32 + wid, s_sp0)

                        @pl.when((si + 1) % 2 == 1)
                        def _():
                            stage_in(1, (si + 1) * 32 + wid, s_sp1)

                if have_xw:
                    @pl.when((tin == 1) & (si >= 1) &
                             (si + 1 < nsup) &
                             ((si + 1) * 32 + wid < gsup_n))
                    def _():
                        @pl.when((si + 1) % 2 == 0)
                        def _():
                            stage_xw(0, (si + 1) * 32 + wid, s_x0)

                        @pl.when((si + 1) % 2 == 1)
                        def _():
                            stage_xw(1, (si + 1) * 32 + wid, s_x1)

                srow = (si % 2) * _SPN + tin * _CH
                for j in range(_CH):
                    v = sp_v[srow + j]
                    idx_v[gb, pl.ds(j * 16, 16)] = \
                        ((v >> 1) << 5) + (v & 1) + lane2

                @pl.when(gb == 0)
                def _():
                    pltpu.async_copy(t_h.at[idx_v.at[0]],
                                     g_v.at[pl.ds(0, gl)], s_g0)

                @pl.when(gb == 1)
                def _():
                    pltpu.async_copy(t_h.at[idx_v.at[1]],
                                     g_v.at[pl.ds(gl, gl)], s_g1)

            u = t - 1
            usi, uin = u // _SUP, u % _SUP
            ugsup = usi * 32 + wid

            @pl.when((t >= 1) & (ugsup < gsup_n))
            def _():
                ub = u % 2
                ob = usi % 2

                @pl.when(ub == 0)
                def _():
                    pltpu.make_async_copy(t_h.at[idx_v.at[0]],
                                          g_v.at[pl.ds(0, gl)], s_g0).wait()

                @pl.when(ub == 1)
                def _():
                    pltpu.make_async_copy(t_h.at[idx_v.at[1]],
                                          g_v.at[pl.ds(gl, gl)], s_g1).wait()

                if have_xw:
                    @pl.when((uin == 0) & (usi >= 1))
                    def _():
                        @pl.when(usi % 2 == 0)
                        def _():
                            wait_xw(s_x0, 0)

                        @pl.when(usi % 2 == 1)
                        def _():
                            wait_xw(s_x1, _SPN)

                gof = ub * gl
                orow = ob * _SPN + uin * _CH
                for j in range(_CH):
                    for c in range(rc):
                        sl = pl.ds(c * 16, 16)
                        if packed:
                            lo = bias_v[pl.ds(c * 16, 16)]
                            hi = bias_v[pl.ds(64 + c * 16, 16)]
                            for k in range(16):
                                w = g_v[gof + j * 16 + k, sl]
                                lo = lo + lax.bitcast_convert_type(
                                    w << 16, jnp.float32)
                                hi = hi + lax.bitcast_convert_type(
                                    w & jnp.int32(-65536), jnp.float32)
                            out_v[orow + j, pl.ds(c * 16, 16)] = lo
                            out_v[orow + j, pl.ds(64 + c * 16, 16)] = hi
                        else:
                            acc = bias_v[sl]
                            if have_xw:
                                acc = acc + xw_v[orow + j, sl]
                            for k in range(16):
                                acc = acc + g_v[gof + j * 16 + k, sl]
                            out_v[orow + j, sl] = acc

                @pl.when(uin == _SUP - 1)
                def _():
                    dst = ugsup * _SPN

                    @pl.when(ob == 0)
                    def _():
                        pltpu.sync_copy(out_v.at[pl.ds(0, _SPN)],
                                        out_h.at[pl.ds(dst, _SPN)])

                    @pl.when(ob == 1)
                    def _():
                        pltpu.sync_copy(out_v.at[pl.ds(_SPN, _SPN)],
                                        out_h.at[pl.ds(dst, _SPN)])
            return carry

        lax.fori_loop(0, nch + 1, step, 0)

    scratch = [pltpu.VMEM((2 * _SPN, 16), jnp.int32),
               pltpu.VMEM((2, gl), jnp.int32),
               pltpu.VMEM((2 * gl, rw), jnp.int32 if packed else jnp.float32),
               pltpu.VMEM((2 * _SPN, ow), jnp.float32),
               pltpu.VMEM((ow,), jnp.float32)]
    if have_xw:
        scratch.append(pltpu.VMEM((2 * _SPN, rw), jnp.float32))
    nsem = 8 if have_xw else 6
    scratch += [pltpu.SemaphoreType.DMA] * nsem
    args = (table, sp, bias) + ((xw,) if have_xw else ())
    params = (pltpu.CompilerParams(use_tc_tiling_on_sc=False)
              if rw < 128 else None)
    return pl.kernel(body,
                     out_type=jax.ShapeDtypeStruct((_N, ow), jnp.float32),
                     mesh=_SC_MESH,
                     compiler_params=params,
                     scratch_types=scratch)(*args)


def kernel(x, spiral, W0, b0, W1, b1, W2, b2, W_out, b_out):
    sp = spiral.astype(jnp.int32)

    w0b = W0.reshape(_S, 3, 16)
    w1b = W1.reshape(_S, 16, 16)
    wh, wx = W_out[:16, :], W_out[16:, :]
    w2f = (W2 @ wh).reshape(_S, 16, 3)
    w2b = jnp.pad(w2f, ((0, 0), (0, 0), (0, 1)))
    hp = _halves_perm(4096)
    wc0 = _make_wcat_cb(w0b, 16)[:, hp].astype(jnp.bfloat16)
    wc1 = _make_wcat(w1b)[:, hp].astype(jnp.bfloat16)
    wc2 = _make_wcat(w2b)
    wxp = jnp.pad(wx, ((0, 0), (0, 1)))
    wxc = _make_wcat_cb(wxp[None], 4)
    bias0 = jnp.tile(b0, (_B,))
    bias1 = jnp.tile(b1, (_B,))
    bias2 = jnp.tile(jnp.pad(b2 @ wh + b_out, (0, 1)), (_B,))

    xt = jnp.transpose(x.reshape(_B, _N * 3))

    t0 = _transform_p(xt.reshape(_N2, 48), wc0).reshape(_N2 * 32, 64)
    h1 = _gather_sum(t0, sp, bias0, packed=True)
    t1 = _transform_p(h1.reshape(_N2, 256), wc1).reshape(_N2 * 32, 64)
    h2 = _gather_sum(t1, sp, bias1, packed=True)
    t2, xw = _transform2(h2.reshape(_N2, 256), xt.reshape(_N2, 48), wc2, wxc)
    outp = _gather_sum(t2.reshape(_N2 * 32, 32), sp, bias2,
                       xw=xw.reshape(_N, 32))

    return jnp.transpose(outp.reshape(_N, _B, 4)[:, :, :3], (1, 0, 2))
